# Initial kernel scaffold; baseline (speedup 1.0000x reference)
import jax
import jax.numpy as jnp
from jax.experimental import pallas as pl


def kernel(x, W1, b1, Wp, bp, Wo, bo):
    raise NotImplementedError("write your pallas kernel here")


# speedup vs baseline: 11.8421x; 11.8421x over previous
import functools

import jax
import jax.numpy as jnp
from jax.experimental import pallas as pl
from jax.experimental.pallas import tpu as pltpu

_HIDDEN = 128
_D_IN = 28 * 28
_PHASOR_H = 32
_DELTA0 = 7.0
_BLK = 1024


def _fused_kernel(x_ref, W1_ref, b1_ref, Wp_ref, bp_ref, Wo_ref, bo_ref, out_ref):
    x = x_ref[...]
    projected = jnp.dot(x, W1_ref[...],
                        preferred_element_type=jnp.float32) + b1_ref[...]
    x_mean = jnp.mean(projected, axis=-1, keepdims=True)
    h = jax.lax.broadcasted_iota(jnp.int32, (1, _PHASOR_H), 1).astype(
        jnp.float32) + 1.0
    phase = x_mean * (_DELTA0 * h)
    feats = jnp.concatenate([jnp.cos(phase), jnp.sin(phase)], axis=-1)
    temporal = jnp.dot(feats, Wp_ref[...],
                       preferred_element_type=jnp.float32) + bp_ref[...]
    enhanced = projected + temporal
    out_ref[...] = jnp.dot(enhanced, Wo_ref[...],
                           preferred_element_type=jnp.float32) + bo_ref[...]


@functools.partial(jax.jit, static_argnames=())
def kernel(x, W1, b1, Wp, bp, Wo, bo):
    B = x.shape[0]
    x_flat = x.reshape(B, _D_IN)
    n_out = Wo.shape[1]
    grid = (B // _BLK,)
    return pl.pallas_call(
        _fused_kernel,
        grid=grid,
        in_specs=[
            pl.BlockSpec((_BLK, _D_IN), lambda i: (i, 0)),
            pl.BlockSpec((_D_IN, _HIDDEN), lambda i: (0, 0)),
            pl.BlockSpec((1, _HIDDEN), lambda i: (0, 0)),
            pl.BlockSpec((2 * _PHASOR_H, _HIDDEN), lambda i: (0, 0)),
            pl.BlockSpec((1, _HIDDEN), lambda i: (0, 0)),
            pl.BlockSpec((_HIDDEN, n_out), lambda i: (0, 0)),
            pl.BlockSpec((1, n_out), lambda i: (0, 0)),
        ],
        out_specs=pl.BlockSpec((_BLK, n_out), lambda i: (i, 0)),
        out_shape=jax.ShapeDtypeStruct((B, n_out), jnp.float32),
        compiler_params=pltpu.CompilerParams(
            dimension_semantics=("arbitrary",),
        ),
    )(x_flat, W1, b1.reshape(1, -1), Wp, bp.reshape(1, -1),
      Wo, bo.reshape(1, -1))
